# baseline (speedup 1.0000x reference)
import jax
import jax.numpy as jnp
from jax.experimental import pallas as pl
from jax.experimental.pallas import tpu as pltpu

_B = 16
_P = 4096
_S = 256
_PB = 512


def _hist_kernel(x_ref, out_ref, hist_acc):
    i = pl.program_id(0)
    counts = jnp.sum(x_ref[...], axis=2)
    bins = jax.lax.broadcasted_iota(jnp.int32, (1, 1, _S), 2)
    onehot = (counts[:, :, None] == bins).astype(jnp.int32)
    part = jnp.sum(onehot, axis=1)

    @pl.when(i == 0)
    def _init():
        hist_acc[...] = part

    @pl.when(i > 0)
    def _accum():
        hist_acc[...] += part

    @pl.when(i == pl.num_programs(0) - 1)
    def _finish():
        h = hist_acc[...].astype(jnp.float32)
        out_ref[...] = h / jnp.sum(h, axis=1, keepdims=True)


def kernel(inputs):
    x = inputs.reshape(_B, _P, _S)
    probs = pl.pallas_call(
        _hist_kernel,
        grid=(_P // _PB,),
        in_specs=[pl.BlockSpec((_B, _PB, _S), lambda i: (0, i, 0))],
        out_specs=pl.BlockSpec((_B, _S), lambda i: (0, 0)),
        out_shape=jax.ShapeDtypeStruct((_B, _S), jnp.float32),
        scratch_shapes=[pltpu.VMEM((_B, _S), jnp.int32)],
    )(x)
    return ((probs,),)
